# baseline (speedup 1.0000x reference)
import functools

import jax
import jax.numpy as jnp
from jax import lax
from jax.experimental import pallas as pl
from jax.experimental.pallas import tpu as pltpu
from jax.experimental.pallas import tpu_sc as plsc

N = 10000
F_IN = 128
H = 4
C = 16
HC = H * C
L = 52
EL = 256

N_PAD = 10240
ROW_W = 68
NC, NS = 2, 16
NW = NC * NS
K = 128
E_TOT = 320000 + N
NPAIR = -(-E_TOT // (NW * K * 2))
CH = 2 * NPAIR
E_W = CH * K
E_PAD = NW * E_W
DUMMY = N
RPT = N_PAD // NS
MB = 512


def _proj_body(x_ref, wl_ref, bl_ref, wr_ref, br_ref, xl_ref, xr_ref):
    xb = x_ref[...]
    xl = jnp.dot(xb, wl_ref[...], preferred_element_type=jnp.float32) + bl_ref[...]
    xr = jnp.dot(xb, wr_ref[...], preferred_element_type=jnp.float32) + br_ref[...]
    xl_ref[...] = xl.astype(jnp.bfloat16)
    xr_ref[...] = xr.astype(jnp.bfloat16)


def _project(x, Wl, bl, Wr, br):
    return pl.pallas_call(
        _proj_body,
        grid=(N_PAD // MB,),
        in_specs=[
            pl.BlockSpec((MB, F_IN), lambda i: (i, 0)),
            pl.BlockSpec((F_IN, HC), lambda i: (0, 0)),
            pl.BlockSpec((1, HC), lambda i: (0, 0)),
            pl.BlockSpec((F_IN, HC), lambda i: (0, 0)),
            pl.BlockSpec((1, HC), lambda i: (0, 0)),
        ],
        out_specs=[
            pl.BlockSpec((MB, HC), lambda i: (i, 0)),
            pl.BlockSpec((MB, HC), lambda i: (i, 0)),
        ],
        out_shape=[
            jax.ShapeDtypeStruct((N_PAD, HC), jnp.bfloat16),
            jax.ShapeDtypeStruct((N_PAD, HC), jnp.bfloat16),
        ],
    )(x, Wl, bl.reshape(1, HC), Wr, br.reshape(1, HC))


def _label_call(label_args):
    full = lambda shape: pl.BlockSpec(shape, lambda: tuple(0 for _ in shape))
    return pl.pallas_call(
        _label_math,
        in_specs=[full((1, 64)), full((EL, 1)), full((1, EL)), full((EL, 1)),
                  full((EL, 1)), full((1, EL)), full((64, 64)), full((1, 64)),
                  full((64, 64)), full((1, 64)), full((64, 64)), full((1, 64))],
        out_specs=[full((64, 64)), full((1, 64))],
        out_shape=[jax.ShapeDtypeStruct((64, 64), jnp.float32),
                   jax.ShapeDtypeStruct((1, 64), jnp.float32)],
    )(*label_args)


def _label_math(lx_ref, lsrc_c_ref, ldst_r_ref, ldst_c_ref, lew_c_ref, lew_r_ref,
                w1_ref, b1_ref, w2_ref, b2_ref, fw_ref, fb_ref, q_ref, qb_ref):
    f32 = jnp.float32
    iota0_64x256 = lax.broadcasted_iota(jnp.int32, (64, 256), 0)
    iota1_256x64 = lax.broadcasted_iota(jnp.int32, (256, 64), 1)
    ohT_dst = (iota0_64x256 == ldst_r_ref[...]).astype(f32)
    oh_src = (iota1_256x64 == lsrc_c_ref[...]).astype(f32)
    oh_dst = (iota1_256x64 == ldst_c_ref[...]).astype(f32)
    ew_c = lew_c_ref[...]
    wadj = jnp.dot(ohT_dst, ew_c * oh_src, preferred_element_type=f32)
    deg_c = jnp.dot(ohT_dst, ew_c, preferred_element_type=f32) + 1.0
    deg_r = jnp.dot(lew_r_ref[...], oh_dst, preferred_element_type=f32) + 1.0
    dinv_c = lax.rsqrt(deg_c)
    dinv_r = lax.rsqrt(deg_r)
    i0 = lax.broadcasted_iota(jnp.int32, (64, 64), 0)
    i1 = lax.broadcasted_iota(jnp.int32, (64, 64), 1)
    eye = (i0 == i1).astype(f32)
    m = dinv_c * (wadj + eye) * dinv_r

    lxv = lx_ref[...]
    lx0 = eye * (lxv / jnp.sum(lxv))

    def elu(v):
        return jnp.where(v > 0, v, jnp.exp(v) - 1.0)

    h1 = elu(jnp.dot(m, jnp.dot(lx0, w1_ref[...], preferred_element_type=f32),
                     preferred_element_type=f32) + b1_ref[...])
    h2 = elu(jnp.dot(m, jnp.dot(h1, w2_ref[...], preferred_element_type=f32),
                     preferred_element_type=f32) + b2_ref[...])
    q_ref[...] = jnp.dot(fw_ref[...], h2, preferred_element_type=f32)
    qb_ref[...] = jnp.dot(fb_ref[...], h2, preferred_element_type=f32)


def _label_args(label_x, label_edge_index, label_edge_weights,
                gcn1_W, gcn1_b, gcn2_W, gcn2_b, fin_W, fin_b):
    f32 = jnp.float32
    lx = jnp.zeros((1, 64), f32).at[0, :L].set(label_x)
    lsrc = label_edge_index[0].astype(jnp.int32)
    ldst = label_edge_index[1].astype(jnp.int32)
    lsrc_c = lsrc.reshape(EL, 1)
    ldst_r = ldst.reshape(1, EL)
    ldst_c = ldst.reshape(EL, 1)
    lew_c = label_edge_weights.reshape(EL, 1)
    lew_r = label_edge_weights.reshape(1, EL)
    w1 = jnp.zeros((64, 64), f32).at[:L, :].set(gcn1_W)
    b1 = jnp.zeros((1, 64), f32).at[0, :].set(gcn1_b)
    w2 = jnp.zeros((64, 64), f32).at[:, :L].set(gcn2_W)
    b2 = jnp.zeros((1, 64), f32).at[0, :L].set(gcn2_b)
    fw = jnp.zeros((64, 64), f32).at[:, :L].set(fin_W)
    fb = jnp.zeros((1, 64), f32).at[0, :L].set(fin_b)
    return (lx, lsrc_c, ldst_r, ldst_c, lew_c, lew_r,
            w1, b1, w2, b2, fw, fb)


def _sc_body(src_hbm, dst_hbm, xl_hbm, xr_hbm, att_hbm, zeros_hbm, out_hbm,
             src_a, dg_a, ds_a, xl_a, xr_a, co_a,
             src_b, dg_b, ds_b, xl_b, xr_b, co_b,
             acc, att_v, gsem_a, gsem_b, ssem_a, ssem_b, isem):
    f32 = jnp.float32
    cid = lax.axis_index("c")
    sid = lax.axis_index("s")
    wid = cid * NS + sid

    pltpu.sync_copy(zeros_hbm, acc.at[pl.ds(sid * RPT, RPT)])
    pltpu.sync_copy(att_hbm, att_v)
    plsc.subcore_barrier()

    att_vecs = [att_v[h, :] for h in range(H)]
    io16 = lax.iota(jnp.int32, 16)
    zeros16 = jnp.zeros((16,), f32)
    p8, p4, p2, p1 = (io16 ^ k for k in (8, 4, 2, 1))
    m_lt4, m_lt8, m_lt12 = io16 < 4, io16 < 8, io16 < 12
    bidx = [io16 * 0 + (4 * h) for h in range(H)]
    didx = (io16 & 3) * 4
    didx_col = 64 + (io16 & 3)

    def g(v, p):
        return v.at[p].get(mode="promise_in_bounds")

    def two_heads(ref, i, k):
        return plsc.unpack(ref[i, pl.ds(32 * k, 32)],
                           format=plsc.PackFormat.INTERLEAVED,
                           preferred_element_type=jnp.float32)

    def compute(xl_v, xr_v, contrib):
        @plsc.parallel_loop(0, K, unroll=8)
        def edge_body(i):
            xs, rs, ss = [], [], []
            for k in range(2):
                x0, x1 = two_heads(xl_v, i, k)
                r0, r1 = two_heads(xr_v, i, k)
                xs.extend((x0, x1))
                rs.extend((r0, r1))
            for h in range(H):
                xh = xs[h]
                t = xh + rs[h]
                e = jnp.maximum(t, 0.2 * t)
                p = e * att_vecs[h]
                s = p + g(p, p8)
                ss.append(s + g(s, p4))
            m = jnp.where(m_lt4, ss[0],
                          jnp.where(m_lt8, ss[1],
                                    jnp.where(m_lt12, ss[2], ss[3])))
            m = m + g(m, p2)
            m = m + g(m, p1)
            em = jnp.exp(m)
            for h in range(H):
                contrib[i, pl.ds(h * 16, 16)] = g(em, bidx[h]) * xs[h]
            row_idx = jnp.full((16,), i, jnp.int32)
            plsc.store_scatter(contrib, [row_idx, didx_col], g(em, didx),
                               mask=m_lt4)

    def load_idx(c, src_v, dg_v, ds_v, sync):
        base = c * K
        i1 = pltpu.async_copy(src_hbm.at[wid, pl.ds(base, K)], src_v, isem)
        i2 = pltpu.async_copy(dst_hbm.at[wid, pl.ds(base, K)], dg_v, isem)
        i3 = pltpu.async_copy(dst_hbm.at[wid, pl.ds(base, K)], ds_v, isem)
        i1.wait()
        i2.wait()
        i3.wait()

    def issue_gathers(src_v, dg_v, xl_v, xr_v, gsem):
        pltpu.async_copy(xl_hbm.at[src_v], xl_v, gsem)
        pltpu.async_copy(xr_hbm.at[dg_v], xr_v, gsem)

    load_idx(0, src_a, dg_a, ds_a, True)
    issue_gathers(src_a, dg_a, xl_a, xr_a, gsem_a)
    load_idx(1, src_b, dg_b, ds_b, True)
    issue_gathers(src_b, dg_b, xl_b, xr_b, gsem_b)

    def phase(j, off, src_v, dg_v, ds_v, xl_v, xr_v, contrib, gsem, ssem):
        pltpu.make_async_copy(xl_hbm.at[src_v], xl_v, gsem).wait()
        pltpu.make_async_copy(xr_hbm.at[dg_v], xr_v, gsem).wait()

        @pl.when(j > 0)
        def _():
            pltpu.make_async_copy(contrib, acc.at[ds_v], ssem).wait()

        compute(xl_v, xr_v, contrib)
        pltpu.async_copy(contrib, acc.at[ds_v], ssem, add=True)

        @pl.when(j < NPAIR - 1)
        def _():
            nc = 2 * j + 2 + off
            load_idx(nc, src_v, dg_v, ds_v, False)
            issue_gathers(src_v, dg_v, xl_v, xr_v, gsem)

    def pair_body(j, carry):
        phase(j, 0, src_a, dg_a, ds_a, xl_a, xr_a, co_a, gsem_a, ssem_a)
        phase(j, 1, src_b, dg_b, ds_b, xl_b, xr_b, co_b, gsem_b, ssem_b)
        return carry

    lax.fori_loop(0, NPAIR, pair_body, 0)
    pltpu.make_async_copy(co_a, acc.at[ds_a], ssem_a).wait()
    pltpu.make_async_copy(co_b, acc.at[ds_b], ssem_b).wait()
    plsc.subcore_barrier()
    pltpu.sync_copy(acc.at[pl.ds(sid * RPT, RPT)],
                    out_hbm.at[cid, pl.ds(sid * RPT, RPT)])


def _sc_aggregate(src_w, dst_w, xl, xr, att, zeros_hbm):
    mesh = plsc.VectorSubcoreMesh(core_axis_name="c", subcore_axis_name="s",
                                  num_cores=NC, num_subcores=NS)
    fn = pl.kernel(
        _sc_body,
        out_type=jax.ShapeDtypeStruct((NC, N_PAD, ROW_W), jnp.float32),
        mesh=mesh,
        scratch_types=(
            2 * [
                pltpu.VMEM((K,), jnp.int32),
                pltpu.VMEM((K,), jnp.int32),
                pltpu.VMEM((K,), jnp.int32),
                pltpu.VMEM((K, HC), jnp.bfloat16),
                pltpu.VMEM((K, HC), jnp.bfloat16),
                pltpu.VMEM((K, ROW_W), jnp.float32),
            ]
            + [
                pltpu.VMEM_SHARED((N_PAD, ROW_W), jnp.float32),
                pltpu.VMEM((H, C), jnp.float32),
                pltpu.SemaphoreType.DMA,
                pltpu.SemaphoreType.DMA,
                pltpu.SemaphoreType.DMA,
                pltpu.SemaphoreType.DMA,
                pltpu.SemaphoreType.DMA,
            ]
        ),
        compiler_params=pltpu.CompilerParams(use_tc_tiling_on_sc=False,
                                             needs_layout_passes=False),
    )
    return fn(src_w, dst_w, xl, xr, att, zeros_hbm)


def _final_body(p0_ref, p1_ref, bias_ref, q_ref, qb_ref, out_ref):
    f32 = jnp.float32
    acc = p0_ref[...] + p1_ref[...]
    num = acc[:, :HC]
    den = acc[:, HC:ROW_W]
    i0 = lax.broadcasted_iota(jnp.int32, (H, HC), 0)
    i1 = lax.broadcasted_iota(jnp.int32, (H, HC), 1) // C
    sel = (i0 == i1).astype(f32)
    den_rep = jnp.dot(den, sel, preferred_element_type=f32)
    gat = num / den_rep + bias_ref[...]
    hg = jnp.where(gat > 0, gat, jnp.exp(gat) - 1.0)
    z = jnp.dot(hg, q_ref[...], preferred_element_type=f32) + qb_ref[...]
    out_ref[...] = jax.nn.sigmoid(lax.slice(z, (0, 0), (MB, L)))


def _finalize(partials, gat_bias, q, qb):
    return pl.pallas_call(
        _final_body,
        grid=(N_PAD // MB,),
        in_specs=[
            pl.BlockSpec((MB, ROW_W), lambda i: (i, 0)),
            pl.BlockSpec((MB, ROW_W), lambda i: (i, 0)),
            pl.BlockSpec((1, HC), lambda i: (0, 0)),
            pl.BlockSpec((64, 64), lambda i: (0, 0)),
            pl.BlockSpec((1, 64), lambda i: (0, 0)),
        ],
        out_specs=pl.BlockSpec((MB, L), lambda i: (i, 0)),
        out_shape=jax.ShapeDtypeStruct((N, L), jnp.float32),
    )(partials[0], partials[1], gat_bias.reshape(1, HC), q, qb)


@jax.jit
def kernel(x, edge_index, label_x, label_edge_index, label_edge_weights,
           gat_Wl, gat_bl, gat_Wr, gat_br, gat_att, gat_bias,
           gcn1_W, gcn1_b, gcn2_W, gcn2_b, fin_W, fin_b):
    f32 = jnp.float32
    j = jnp.arange(HC)
    pidx = (2 * (j // 32) + (j % 2)) * C + (j % 32) // 2
    largs = _label_args(label_x, label_edge_index, label_edge_weights,
                        gcn1_W, gcn1_b, gcn2_W, gcn2_b, fin_W, fin_b)
    xl, xr = _project(x, gat_Wl[:, pidx], gat_bl[pidx],
                      gat_Wr[:, pidx], gat_br[pidx])
    q, qb = _label_call(largs)

    loops = jnp.arange(N, dtype=jnp.int32)
    pad = jnp.full((E_PAD - E_TOT,), DUMMY, jnp.int32)
    src_w = jnp.concatenate([edge_index[0].astype(jnp.int32), loops, pad]).reshape(NW, E_W)
    dst_w = jnp.concatenate([edge_index[1].astype(jnp.int32), loops, pad]).reshape(NW, E_W)

    zeros_hbm = jnp.zeros((RPT, ROW_W), f32)
    partials = _sc_aggregate(src_w, dst_w, xl, xr, gat_att, zeros_hbm)

    return _finalize(partials, gat_bias, q, qb)


# speedup vs baseline: 1.1673x; 1.1673x over previous
import functools

import jax
import jax.numpy as jnp
from jax import lax
from jax.experimental import pallas as pl
from jax.experimental.pallas import tpu as pltpu
from jax.experimental.pallas import tpu_sc as plsc

N = 10000
F_IN = 128
H = 4
C = 16
HC = H * C
L = 52
EL = 256

N_PAD = 10240
ROW_W = 80
NC, NS = 2, 16
NW = NC * NS
K = 128
E_TOT = 320000 + N
NPAIR = -(-E_TOT // (NW * K * 2))
CH = 2 * NPAIR
E_W = CH * K
E_PAD = NW * E_W
DUMMY = N
RPT = N_PAD // NS
MB = 512


def _proj_body(x_ref, wl_ref, bl_ref, wr_ref, br_ref, xl_ref, xr_ref):
    xb = x_ref[...]
    xl = jnp.dot(xb, wl_ref[...], preferred_element_type=jnp.float32) + bl_ref[...]
    xr = jnp.dot(xb, wr_ref[...], preferred_element_type=jnp.float32) + br_ref[...]
    xl_ref[...] = xl.astype(jnp.bfloat16)
    xr_ref[...] = xr.astype(jnp.bfloat16)


def _project(x, Wl, bl, Wr, br):
    return pl.pallas_call(
        _proj_body,
        grid=(N_PAD // MB,),
        in_specs=[
            pl.BlockSpec((MB, F_IN), lambda i: (i, 0)),
            pl.BlockSpec((F_IN, HC), lambda i: (0, 0)),
            pl.BlockSpec((1, HC), lambda i: (0, 0)),
            pl.BlockSpec((F_IN, HC), lambda i: (0, 0)),
            pl.BlockSpec((1, HC), lambda i: (0, 0)),
        ],
        out_specs=[
            pl.BlockSpec((MB, HC), lambda i: (i, 0)),
            pl.BlockSpec((MB, HC), lambda i: (i, 0)),
        ],
        out_shape=[
            jax.ShapeDtypeStruct((N_PAD, HC), jnp.bfloat16),
            jax.ShapeDtypeStruct((N_PAD, HC), jnp.bfloat16),
        ],
    )(x, Wl, bl.reshape(1, HC), Wr, br.reshape(1, HC))


def _label_call(label_args):
    full = lambda shape: pl.BlockSpec(shape, lambda: tuple(0 for _ in shape))
    return pl.pallas_call(
        _label_math,
        in_specs=[full((1, 64)), full((EL, 1)), full((1, EL)), full((EL, 1)),
                  full((EL, 1)), full((1, EL)), full((64, 64)), full((1, 64)),
                  full((64, 64)), full((1, 64)), full((64, 64)), full((1, 64))],
        out_specs=[full((64, 64)), full((1, 64))],
        out_shape=[jax.ShapeDtypeStruct((64, 64), jnp.float32),
                   jax.ShapeDtypeStruct((1, 64), jnp.float32)],
    )(*label_args)


def _label_math(lx_ref, lsrc_c_ref, ldst_r_ref, ldst_c_ref, lew_c_ref, lew_r_ref,
                w1_ref, b1_ref, w2_ref, b2_ref, fw_ref, fb_ref, q_ref, qb_ref):
    f32 = jnp.float32
    iota0_64x256 = lax.broadcasted_iota(jnp.int32, (64, 256), 0)
    iota1_256x64 = lax.broadcasted_iota(jnp.int32, (256, 64), 1)
    ohT_dst = (iota0_64x256 == ldst_r_ref[...]).astype(f32)
    oh_src = (iota1_256x64 == lsrc_c_ref[...]).astype(f32)
    oh_dst = (iota1_256x64 == ldst_c_ref[...]).astype(f32)
    ew_c = lew_c_ref[...]
    wadj = jnp.dot(ohT_dst, ew_c * oh_src, preferred_element_type=f32)
    deg_c = jnp.dot(ohT_dst, ew_c, preferred_element_type=f32) + 1.0
    deg_r = jnp.dot(lew_r_ref[...], oh_dst, preferred_element_type=f32) + 1.0
    dinv_c = lax.rsqrt(deg_c)
    dinv_r = lax.rsqrt(deg_r)
    i0 = lax.broadcasted_iota(jnp.int32, (64, 64), 0)
    i1 = lax.broadcasted_iota(jnp.int32, (64, 64), 1)
    eye = (i0 == i1).astype(f32)
    m = dinv_c * (wadj + eye) * dinv_r

    lxv = lx_ref[...]
    lx0 = eye * (lxv / jnp.sum(lxv))

    def elu(v):
        return jnp.where(v > 0, v, jnp.exp(v) - 1.0)

    h1 = elu(jnp.dot(m, jnp.dot(lx0, w1_ref[...], preferred_element_type=f32),
                     preferred_element_type=f32) + b1_ref[...])
    h2 = elu(jnp.dot(m, jnp.dot(h1, w2_ref[...], preferred_element_type=f32),
                     preferred_element_type=f32) + b2_ref[...])
    q_ref[...] = jnp.dot(fw_ref[...], h2, preferred_element_type=f32)
    qb_ref[...] = jnp.dot(fb_ref[...], h2, preferred_element_type=f32)


def _label_args(label_x, label_edge_index, label_edge_weights,
                gcn1_W, gcn1_b, gcn2_W, gcn2_b, fin_W, fin_b):
    f32 = jnp.float32
    lx = jnp.zeros((1, 64), f32).at[0, :L].set(label_x)
    lsrc = label_edge_index[0].astype(jnp.int32)
    ldst = label_edge_index[1].astype(jnp.int32)
    lsrc_c = lsrc.reshape(EL, 1)
    ldst_r = ldst.reshape(1, EL)
    ldst_c = ldst.reshape(EL, 1)
    lew_c = label_edge_weights.reshape(EL, 1)
    lew_r = label_edge_weights.reshape(1, EL)
    w1 = jnp.zeros((64, 64), f32).at[:L, :].set(gcn1_W)
    b1 = jnp.zeros((1, 64), f32).at[0, :].set(gcn1_b)
    w2 = jnp.zeros((64, 64), f32).at[:, :L].set(gcn2_W)
    b2 = jnp.zeros((1, 64), f32).at[0, :L].set(gcn2_b)
    fw = jnp.zeros((64, 64), f32).at[:, :L].set(fin_W)
    fb = jnp.zeros((1, 64), f32).at[0, :L].set(fin_b)
    return (lx, lsrc_c, ldst_r, ldst_c, lew_c, lew_r,
            w1, b1, w2, b2, fw, fb)


def _sc_body(src_hbm, dst_hbm, xl_hbm, xr_hbm, att_hbm, zeros_hbm, out_hbm,
             src_a, dg_a, ds_a, xl_a, xr_a, co_a,
             src_b, dg_b, ds_b, xl_b, xr_b, co_b,
             acc, att_v, gsem_a, gsem_b, ssem_a, ssem_b, isem):
    f32 = jnp.float32
    cid = lax.axis_index("c")
    sid = lax.axis_index("s")
    wid = cid * NS + sid

    pltpu.sync_copy(zeros_hbm, acc.at[pl.ds(sid * RPT, RPT)])
    pltpu.sync_copy(att_hbm, att_v)
    plsc.subcore_barrier()

    att_vecs = [att_v[h, :] for h in range(H)]
    io16 = lax.iota(jnp.int32, 16)
    zeros16 = jnp.zeros((16,), f32)
    p8, p4, p2, p1 = (io16 ^ k for k in (8, 4, 2, 1))
    m_lt4, m_lt8, m_lt12 = io16 < 4, io16 < 8, io16 < 12
    bidx = [io16 * 0 + (4 * h) for h in range(H)]
    didx = (io16 & 3) * 4

    def g(v, p):
        return v.at[p].get(mode="promise_in_bounds")

    def two_heads(ref, i, k):
        return plsc.unpack(ref[i, pl.ds(32 * k, 32)],
                           format=plsc.PackFormat.INTERLEAVED,
                           preferred_element_type=jnp.float32)

    def compute(xl_v, xr_v, contrib):
        @plsc.parallel_loop(0, K, unroll=8)
        def edge_body(i):
            xs, rs, ss = [], [], []
            for k in range(2):
                x0, x1 = two_heads(xl_v, i, k)
                r0, r1 = two_heads(xr_v, i, k)
                xs.extend((x0, x1))
                rs.extend((r0, r1))
            for h in range(H):
                xh = xs[h]
                t = xh + rs[h]
                e = jnp.maximum(t, 0.2 * t)
                p = e * att_vecs[h]
                s = p + g(p, p8)
                ss.append(s + g(s, p4))
            m = jnp.where(m_lt4, ss[0],
                          jnp.where(m_lt8, ss[1],
                                    jnp.where(m_lt12, ss[2], ss[3])))
            m = m + g(m, p2)
            m = m + g(m, p1)
            em = jnp.exp(m)
            for h in range(H):
                contrib[i, pl.ds(h * 16, 16)] = g(em, bidx[h]) * xs[h]
            contrib[i, pl.ds(64, 16)] = jnp.where(m_lt4, g(em, didx), zeros16)

    def load_idx(c, src_v, dg_v, ds_v, sync):
        base = c * K
        i1 = pltpu.async_copy(src_hbm.at[wid, pl.ds(base, K)], src_v, isem)
        i2 = pltpu.async_copy(dst_hbm.at[wid, pl.ds(base, K)], dg_v, isem)
        i3 = pltpu.async_copy(dst_hbm.at[wid, pl.ds(base, K)], ds_v, isem)
        i1.wait()
        i2.wait()
        i3.wait()

    def issue_gathers(src_v, dg_v, xl_v, xr_v, gsem):
        pltpu.async_copy(xl_hbm.at[src_v], xl_v, gsem)
        pltpu.async_copy(xr_hbm.at[dg_v], xr_v, gsem)

    load_idx(0, src_a, dg_a, ds_a, True)
    issue_gathers(src_a, dg_a, xl_a, xr_a, gsem_a)
    load_idx(1, src_b, dg_b, ds_b, True)
    issue_gathers(src_b, dg_b, xl_b, xr_b, gsem_b)

    def phase(j, off, src_v, dg_v, ds_v, xl_v, xr_v, contrib, gsem, ssem):
        pltpu.make_async_copy(xl_hbm.at[src_v], xl_v, gsem).wait()
        pltpu.make_async_copy(xr_hbm.at[dg_v], xr_v, gsem).wait()

        @pl.when(j > 0)
        def _():
            pltpu.make_async_copy(contrib, acc.at[ds_v], ssem).wait()

        compute(xl_v, xr_v, contrib)
        pltpu.async_copy(contrib, acc.at[ds_v], ssem, add=True)

        @pl.when(j < NPAIR - 1)
        def _():
            nc = 2 * j + 2 + off
            load_idx(nc, src_v, dg_v, ds_v, False)
            issue_gathers(src_v, dg_v, xl_v, xr_v, gsem)

    def pair_body(j, carry):
        phase(j, 0, src_a, dg_a, ds_a, xl_a, xr_a, co_a, gsem_a, ssem_a)
        phase(j, 1, src_b, dg_b, ds_b, xl_b, xr_b, co_b, gsem_b, ssem_b)
        return carry

    lax.fori_loop(0, NPAIR, pair_body, 0)
    pltpu.make_async_copy(co_a, acc.at[ds_a], ssem_a).wait()
    pltpu.make_async_copy(co_b, acc.at[ds_b], ssem_b).wait()
    plsc.subcore_barrier()
    pltpu.sync_copy(acc.at[pl.ds(sid * RPT, RPT)],
                    out_hbm.at[cid, pl.ds(sid * RPT, RPT)])


def _sc_aggregate(src_w, dst_w, xl, xr, att, zeros_hbm):
    mesh = plsc.VectorSubcoreMesh(core_axis_name="c", subcore_axis_name="s",
                                  num_cores=NC, num_subcores=NS)
    fn = pl.kernel(
        _sc_body,
        out_type=jax.ShapeDtypeStruct((NC, N_PAD, ROW_W), jnp.float32),
        mesh=mesh,
        scratch_types=(
            2 * [
                pltpu.VMEM((K,), jnp.int32),
                pltpu.VMEM((K,), jnp.int32),
                pltpu.VMEM((K,), jnp.int32),
                pltpu.VMEM((K, HC), jnp.bfloat16),
                pltpu.VMEM((K, HC), jnp.bfloat16),
                pltpu.VMEM((K, ROW_W), jnp.float32),
            ]
            + [
                pltpu.VMEM_SHARED((N_PAD, ROW_W), jnp.float32),
                pltpu.VMEM((H, C), jnp.float32),
                pltpu.SemaphoreType.DMA,
                pltpu.SemaphoreType.DMA,
                pltpu.SemaphoreType.DMA,
                pltpu.SemaphoreType.DMA,
                pltpu.SemaphoreType.DMA,
            ]
        ),
        compiler_params=pltpu.CompilerParams(use_tc_tiling_on_sc=False,
                                             needs_layout_passes=False),
    )
    return fn(src_w, dst_w, xl, xr, att, zeros_hbm)


def _final_body(p0_ref, p1_ref, bias_ref, q_ref, qb_ref, out_ref):
    f32 = jnp.float32
    acc = p0_ref[...] + p1_ref[...]
    num = acc[:, :HC]
    den = acc[:, HC:ROW_W]
    i0 = lax.broadcasted_iota(jnp.int32, (16, HC), 0)
    i1 = lax.broadcasted_iota(jnp.int32, (16, HC), 1) // C
    sel = (i0 == i1).astype(f32)
    den_rep = jnp.dot(den, sel, preferred_element_type=f32)
    gat = num / den_rep + bias_ref[...]
    hg = jnp.where(gat > 0, gat, jnp.exp(gat) - 1.0)
    z = jnp.dot(hg, q_ref[...], preferred_element_type=f32) + qb_ref[...]
    out_ref[...] = jax.nn.sigmoid(lax.slice(z, (0, 0), (MB, L)))


def _finalize(partials, gat_bias, q, qb):
    return pl.pallas_call(
        _final_body,
        grid=(N_PAD // MB,),
        in_specs=[
            pl.BlockSpec((MB, ROW_W), lambda i: (i, 0)),
            pl.BlockSpec((MB, ROW_W), lambda i: (i, 0)),
            pl.BlockSpec((1, HC), lambda i: (0, 0)),
            pl.BlockSpec((64, 64), lambda i: (0, 0)),
            pl.BlockSpec((1, 64), lambda i: (0, 0)),
        ],
        out_specs=pl.BlockSpec((MB, L), lambda i: (i, 0)),
        out_shape=jax.ShapeDtypeStruct((N, L), jnp.float32),
    )(partials[0], partials[1], gat_bias.reshape(1, HC), q, qb)


@jax.jit
def kernel(x, edge_index, label_x, label_edge_index, label_edge_weights,
           gat_Wl, gat_bl, gat_Wr, gat_br, gat_att, gat_bias,
           gcn1_W, gcn1_b, gcn2_W, gcn2_b, fin_W, fin_b):
    f32 = jnp.float32
    j = jnp.arange(HC)
    pidx = (2 * (j // 32) + (j % 2)) * C + (j % 32) // 2
    largs = _label_args(label_x, label_edge_index, label_edge_weights,
                        gcn1_W, gcn1_b, gcn2_W, gcn2_b, fin_W, fin_b)
    xl, xr = _project(x, gat_Wl[:, pidx], gat_bl[pidx],
                      gat_Wr[:, pidx], gat_br[pidx])
    q, qb = _label_call(largs)

    loops = jnp.arange(N, dtype=jnp.int32)
    pad = jnp.full((E_PAD - E_TOT,), DUMMY, jnp.int32)
    src_w = jnp.concatenate([edge_index[0].astype(jnp.int32), loops, pad]).reshape(NW, E_W)
    dst_w = jnp.concatenate([edge_index[1].astype(jnp.int32), loops, pad]).reshape(NW, E_W)

    zeros_hbm = jnp.zeros((RPT, ROW_W), f32)
    partials = _sc_aggregate(src_w, dst_w, xl, xr, gat_att, zeros_hbm)

    return _finalize(partials, gat_bias, q, qb)


# speedup vs baseline: 1.1716x; 1.0037x over previous
import functools

import jax
import jax.numpy as jnp
from jax import lax
from jax.experimental import pallas as pl
from jax.experimental.pallas import tpu as pltpu
from jax.experimental.pallas import tpu_sc as plsc

N = 10000
F_IN = 128
H = 4
C = 16
HC = H * C
L = 52
EL = 256

N_PAD = 10240
ROW_W = 80
NC, NS = 2, 16
NW = NC * NS
K = 128
E_TOT = 320000 + N
NTRI = -(-E_TOT // (NW * K * 3))
CH = 3 * NTRI
E_W = CH * K
E_PAD = NW * E_W
DUMMY = N
RPT = N_PAD // NS
MB = 512


def _proj_body(x_ref, wl_ref, bl_ref, wr_ref, br_ref, xl_ref, xr_ref):
    xb = x_ref[...]
    xl = jnp.dot(xb, wl_ref[...], preferred_element_type=jnp.float32) + bl_ref[...]
    xr = jnp.dot(xb, wr_ref[...], preferred_element_type=jnp.float32) + br_ref[...]
    xl_ref[...] = xl.astype(jnp.bfloat16)
    xr_ref[...] = xr.astype(jnp.bfloat16)


def _project(x, Wl, bl, Wr, br):
    return pl.pallas_call(
        _proj_body,
        grid=(N_PAD // MB,),
        in_specs=[
            pl.BlockSpec((MB, F_IN), lambda i: (i, 0)),
            pl.BlockSpec((F_IN, HC), lambda i: (0, 0)),
            pl.BlockSpec((1, HC), lambda i: (0, 0)),
            pl.BlockSpec((F_IN, HC), lambda i: (0, 0)),
            pl.BlockSpec((1, HC), lambda i: (0, 0)),
        ],
        out_specs=[
            pl.BlockSpec((MB, HC), lambda i: (i, 0)),
            pl.BlockSpec((MB, HC), lambda i: (i, 0)),
        ],
        out_shape=[
            jax.ShapeDtypeStruct((N_PAD, HC), jnp.bfloat16),
            jax.ShapeDtypeStruct((N_PAD, HC), jnp.bfloat16),
        ],
    )(x, Wl, bl.reshape(1, HC), Wr, br.reshape(1, HC))


def _label_call(label_args):
    full = lambda shape: pl.BlockSpec(shape, lambda: tuple(0 for _ in shape))
    return pl.pallas_call(
        _label_math,
        in_specs=[full((1, 64)), full((EL, 1)), full((1, EL)), full((EL, 1)),
                  full((EL, 1)), full((1, EL)), full((64, 64)), full((1, 64)),
                  full((64, 64)), full((1, 64)), full((64, 64)), full((1, 64))],
        out_specs=[full((64, 64)), full((1, 64))],
        out_shape=[jax.ShapeDtypeStruct((64, 64), jnp.float32),
                   jax.ShapeDtypeStruct((1, 64), jnp.float32)],
    )(*label_args)


def _label_math(lx_ref, lsrc_c_ref, ldst_r_ref, ldst_c_ref, lew_c_ref, lew_r_ref,
                w1_ref, b1_ref, w2_ref, b2_ref, fw_ref, fb_ref, q_ref, qb_ref):
    f32 = jnp.float32
    iota0_64x256 = lax.broadcasted_iota(jnp.int32, (64, 256), 0)
    iota1_256x64 = lax.broadcasted_iota(jnp.int32, (256, 64), 1)
    ohT_dst = (iota0_64x256 == ldst_r_ref[...]).astype(f32)
    oh_src = (iota1_256x64 == lsrc_c_ref[...]).astype(f32)
    oh_dst = (iota1_256x64 == ldst_c_ref[...]).astype(f32)
    ew_c = lew_c_ref[...]
    wadj = jnp.dot(ohT_dst, ew_c * oh_src, preferred_element_type=f32)
    deg_c = jnp.dot(ohT_dst, ew_c, preferred_element_type=f32) + 1.0
    deg_r = jnp.dot(lew_r_ref[...], oh_dst, preferred_element_type=f32) + 1.0
    dinv_c = lax.rsqrt(deg_c)
    dinv_r = lax.rsqrt(deg_r)
    i0 = lax.broadcasted_iota(jnp.int32, (64, 64), 0)
    i1 = lax.broadcasted_iota(jnp.int32, (64, 64), 1)
    eye = (i0 == i1).astype(f32)
    m = dinv_c * (wadj + eye) * dinv_r

    lxv = lx_ref[...]
    lx0 = eye * (lxv / jnp.sum(lxv))

    def elu(v):
        return jnp.where(v > 0, v, jnp.exp(v) - 1.0)

    h1 = elu(jnp.dot(m, jnp.dot(lx0, w1_ref[...], preferred_element_type=f32),
                     preferred_element_type=f32) + b1_ref[...])
    h2 = elu(jnp.dot(m, jnp.dot(h1, w2_ref[...], preferred_element_type=f32),
                     preferred_element_type=f32) + b2_ref[...])
    q_ref[...] = jnp.dot(fw_ref[...], h2, preferred_element_type=f32)
    qb_ref[...] = jnp.dot(fb_ref[...], h2, preferred_element_type=f32)


def _label_args(label_x, label_edge_index, label_edge_weights,
                gcn1_W, gcn1_b, gcn2_W, gcn2_b, fin_W, fin_b):
    f32 = jnp.float32
    lx = jnp.zeros((1, 64), f32).at[0, :L].set(label_x)
    lsrc = label_edge_index[0].astype(jnp.int32)
    ldst = label_edge_index[1].astype(jnp.int32)
    lsrc_c = lsrc.reshape(EL, 1)
    ldst_r = ldst.reshape(1, EL)
    ldst_c = ldst.reshape(EL, 1)
    lew_c = label_edge_weights.reshape(EL, 1)
    lew_r = label_edge_weights.reshape(1, EL)
    w1 = jnp.zeros((64, 64), f32).at[:L, :].set(gcn1_W)
    b1 = jnp.zeros((1, 64), f32).at[0, :].set(gcn1_b)
    w2 = jnp.zeros((64, 64), f32).at[:, :L].set(gcn2_W)
    b2 = jnp.zeros((1, 64), f32).at[0, :L].set(gcn2_b)
    fw = jnp.zeros((64, 64), f32).at[:, :L].set(fin_W)
    fb = jnp.zeros((1, 64), f32).at[0, :L].set(fin_b)
    return (lx, lsrc_c, ldst_r, ldst_c, lew_c, lew_r,
            w1, b1, w2, b2, fw, fb)


def _sc_body(src_hbm, dst_hbm, xl_hbm, xr_hbm, att_hbm, zeros_hbm, out_hbm,
             src_a, dg_a, ds_a, xl_a, xr_a, co_a,
             src_b, dg_b, ds_b, xl_b, xr_b, co_b,
             src_c, dg_c, ds_c, xl_c, xr_c, co_c,
             acc, att_v, gsem_a, gsem_b, gsem_c, ssem_a, ssem_b, ssem_c, isem):
    f32 = jnp.float32
    cid = lax.axis_index("c")
    sid = lax.axis_index("s")
    wid = cid * NS + sid

    pltpu.sync_copy(zeros_hbm, acc.at[pl.ds(sid * RPT, RPT)])
    pltpu.sync_copy(att_hbm, att_v)
    plsc.subcore_barrier()

    att_vecs = [att_v[h, :] for h in range(H)]
    io16 = lax.iota(jnp.int32, 16)
    zeros16 = jnp.zeros((16,), f32)
    p8, p4, p2, p1 = (io16 ^ k for k in (8, 4, 2, 1))
    m_lt4, m_lt8, m_lt12 = io16 < 4, io16 < 8, io16 < 12
    bidx = [io16 * 0 + (4 * h) for h in range(H)]
    didx = (io16 & 3) * 4

    def g(v, p):
        return v.at[p].get(mode="promise_in_bounds")

    def two_heads(ref, i, k):
        return plsc.unpack(ref[i, pl.ds(32 * k, 32)],
                           format=plsc.PackFormat.INTERLEAVED,
                           preferred_element_type=jnp.float32)

    def compute(xl_v, xr_v, contrib):
        @plsc.parallel_loop(0, K, unroll=8)
        def edge_body(i):
            xs, rs, ss = [], [], []
            for k in range(2):
                x0, x1 = two_heads(xl_v, i, k)
                r0, r1 = two_heads(xr_v, i, k)
                xs.extend((x0, x1))
                rs.extend((r0, r1))
            for h in range(H):
                xh = xs[h]
                t = xh + rs[h]
                e = jnp.maximum(t, 0.2 * t)
                p = e * att_vecs[h]
                s = p + g(p, p8)
                ss.append(s + g(s, p4))
            m = jnp.where(m_lt4, ss[0],
                          jnp.where(m_lt8, ss[1],
                                    jnp.where(m_lt12, ss[2], ss[3])))
            m = m + g(m, p2)
            m = m + g(m, p1)
            em = jnp.exp(m)
            for h in range(H):
                contrib[i, pl.ds(h * 16, 16)] = g(em, bidx[h]) * xs[h]
            contrib[i, pl.ds(64, 16)] = jnp.where(m_lt4, g(em, didx), zeros16)

    def load_idx(c, src_v, dg_v, ds_v, sync):
        base = c * K
        i1 = pltpu.async_copy(src_hbm.at[wid, pl.ds(base, K)], src_v, isem)
        i2 = pltpu.async_copy(dst_hbm.at[wid, pl.ds(base, K)], dg_v, isem)
        i3 = pltpu.async_copy(dst_hbm.at[wid, pl.ds(base, K)], ds_v, isem)
        i1.wait()
        i2.wait()
        i3.wait()

    def issue_gathers(src_v, dg_v, xl_v, xr_v, gsem):
        pltpu.async_copy(xl_hbm.at[src_v], xl_v, gsem)
        pltpu.async_copy(xr_hbm.at[dg_v], xr_v, gsem)

    load_idx(0, src_a, dg_a, ds_a, True)
    issue_gathers(src_a, dg_a, xl_a, xr_a, gsem_a)
    load_idx(1, src_b, dg_b, ds_b, True)
    issue_gathers(src_b, dg_b, xl_b, xr_b, gsem_b)
    load_idx(2, src_c, dg_c, ds_c, True)
    issue_gathers(src_c, dg_c, xl_c, xr_c, gsem_c)

    def phase(j, off, src_v, dg_v, ds_v, xl_v, xr_v, contrib, gsem, ssem):
        pltpu.make_async_copy(xl_hbm.at[src_v], xl_v, gsem).wait()
        pltpu.make_async_copy(xr_hbm.at[dg_v], xr_v, gsem).wait()

        @pl.when(j > 0)
        def _():
            pltpu.make_async_copy(contrib, acc.at[ds_v], ssem).wait()

        compute(xl_v, xr_v, contrib)
        pltpu.async_copy(contrib, acc.at[ds_v], ssem, add=True)

        @pl.when(j < NTRI - 1)
        def _():
            nc = 3 * j + 3 + off
            load_idx(nc, src_v, dg_v, ds_v, False)
            issue_gathers(src_v, dg_v, xl_v, xr_v, gsem)

    def tri_body(j, carry):
        phase(j, 0, src_a, dg_a, ds_a, xl_a, xr_a, co_a, gsem_a, ssem_a)
        phase(j, 1, src_b, dg_b, ds_b, xl_b, xr_b, co_b, gsem_b, ssem_b)
        phase(j, 2, src_c, dg_c, ds_c, xl_c, xr_c, co_c, gsem_c, ssem_c)
        return carry

    lax.fori_loop(0, NTRI, tri_body, 0)
    pltpu.make_async_copy(co_a, acc.at[ds_a], ssem_a).wait()
    pltpu.make_async_copy(co_b, acc.at[ds_b], ssem_b).wait()
    pltpu.make_async_copy(co_c, acc.at[ds_c], ssem_c).wait()
    plsc.subcore_barrier()
    pltpu.sync_copy(acc.at[pl.ds(sid * RPT, RPT)],
                    out_hbm.at[cid, pl.ds(sid * RPT, RPT)])


def _sc_aggregate(src_w, dst_w, xl, xr, att, zeros_hbm):
    mesh = plsc.VectorSubcoreMesh(core_axis_name="c", subcore_axis_name="s",
                                  num_cores=NC, num_subcores=NS)
    fn = pl.kernel(
        _sc_body,
        out_type=jax.ShapeDtypeStruct((NC, N_PAD, ROW_W), jnp.float32),
        mesh=mesh,
        scratch_types=(
            3 * [
                pltpu.VMEM((K,), jnp.int32),
                pltpu.VMEM((K,), jnp.int32),
                pltpu.VMEM((K,), jnp.int32),
                pltpu.VMEM((K, HC), jnp.bfloat16),
                pltpu.VMEM((K, HC), jnp.bfloat16),
                pltpu.VMEM((K, ROW_W), jnp.float32),
            ]
            + [
                pltpu.VMEM_SHARED((N_PAD, ROW_W), jnp.float32),
                pltpu.VMEM((H, C), jnp.float32),
            ]
            + 7 * [pltpu.SemaphoreType.DMA]
        ),
        compiler_params=pltpu.CompilerParams(use_tc_tiling_on_sc=False,
                                             needs_layout_passes=False),
    )
    return fn(src_w, dst_w, xl, xr, att, zeros_hbm)


def _final_body(p0_ref, p1_ref, bias_ref, q_ref, qb_ref, out_ref):
    f32 = jnp.float32
    acc = p0_ref[...] + p1_ref[...]
    num = acc[:, :HC]
    den = acc[:, HC:ROW_W]
    i0 = lax.broadcasted_iota(jnp.int32, (16, HC), 0)
    i1 = lax.broadcasted_iota(jnp.int32, (16, HC), 1) // C
    sel = (i0 == i1).astype(f32)
    den_rep = jnp.dot(den, sel, preferred_element_type=f32)
    gat = num / den_rep + bias_ref[...]
    hg = jnp.where(gat > 0, gat, jnp.exp(gat) - 1.0)
    z = jnp.dot(hg, q_ref[...], preferred_element_type=f32) + qb_ref[...]
    out_ref[...] = jax.nn.sigmoid(lax.slice(z, (0, 0), (MB, L)))


def _finalize(partials, gat_bias, q, qb):
    return pl.pallas_call(
        _final_body,
        grid=(N_PAD // MB,),
        in_specs=[
            pl.BlockSpec((MB, ROW_W), lambda i: (i, 0)),
            pl.BlockSpec((MB, ROW_W), lambda i: (i, 0)),
            pl.BlockSpec((1, HC), lambda i: (0, 0)),
            pl.BlockSpec((64, 64), lambda i: (0, 0)),
            pl.BlockSpec((1, 64), lambda i: (0, 0)),
        ],
        out_specs=pl.BlockSpec((MB, L), lambda i: (i, 0)),
        out_shape=jax.ShapeDtypeStruct((N, L), jnp.float32),
    )(partials[0], partials[1], gat_bias.reshape(1, HC), q, qb)


@jax.jit
def kernel(x, edge_index, label_x, label_edge_index, label_edge_weights,
           gat_Wl, gat_bl, gat_Wr, gat_br, gat_att, gat_bias,
           gcn1_W, gcn1_b, gcn2_W, gcn2_b, fin_W, fin_b):
    f32 = jnp.float32
    j = jnp.arange(HC)
    pidx = (2 * (j // 32) + (j % 2)) * C + (j % 32) // 2
    largs = _label_args(label_x, label_edge_index, label_edge_weights,
                        gcn1_W, gcn1_b, gcn2_W, gcn2_b, fin_W, fin_b)
    xl, xr = _project(x, gat_Wl[:, pidx], gat_bl[pidx],
                      gat_Wr[:, pidx], gat_br[pidx])
    q, qb = _label_call(largs)

    loops = jnp.arange(N, dtype=jnp.int32)
    pad = jnp.full((E_PAD - E_TOT,), DUMMY, jnp.int32)
    src_w = jnp.concatenate([edge_index[0].astype(jnp.int32), loops, pad]).reshape(NW, E_W)
    dst_w = jnp.concatenate([edge_index[1].astype(jnp.int32), loops, pad]).reshape(NW, E_W)

    zeros_hbm = jnp.zeros((RPT, ROW_W), f32)
    partials = _sc_aggregate(src_w, dst_w, xl, xr, gat_att, zeros_hbm)

    return _finalize(partials, gat_bias, q, qb)
